# baseline (speedup 1.0000x reference)
import functools

import jax
import jax.numpy as jnp
from jax import lax
from jax.experimental import pallas as pl
from jax.experimental.pallas import tpu as pltpu
from jax.experimental.pallas import tpu_sc as plsc

N = 10000
E = 320000
F = 128
G = 32
K = 3
BN_EPS = 1e-5

NC, NS = 2, 16
NW = NC * NS
NPAD = 10240
CH = 128
CPW = 80
CPW0 = 160
CPW1 = 0
EPAD = NW * CPW * CH
RPT = NPAD // NS

_mesh = plsc.VectorSubcoreMesh(core_axis_name="c", subcore_axis_name="s")


def _zero_fill(buf, nrows, ncols):
  zero = jnp.zeros((16,), jnp.float32)

  def row(i, _):
    for l in range(ncols // 16):
      buf[i, pl.ds(l * 16, 16)] = zero
    return 0

  lax.fori_loop(0, nrows, row, 0)


def _zero_acc_slice(zbuf, acc, sid):
  full, rem = RPT // CH, RPT % CH
  for t in range(full):
    pltpu.sync_copy(zbuf, acc.at[pl.ds(sid * RPT + t * CH, CH)])
  if rem:
    pltpu.sync_copy(zbuf.at[pl.ds(0, rem)],
                    acc.at[pl.ds(sid * RPT + full * CH, rem)])


@functools.partial(
    pl.kernel,
    out_type=jax.ShapeDtypeStruct((NC, NPAD, F), jnp.float32),
    mesh=_mesh,
    scratch_types=[
        pltpu.VMEM((CH,), jnp.int32),
        pltpu.VMEM((CH, F), jnp.float32),
        pltpu.VMEM_SHARED((NPAD, F), jnp.float32),
    ],
)
def _deg_kernel(dst_hbm, out_hbm, idxb, ones_v, hist):
  cid = lax.axis_index("c")
  sid = lax.axis_index("s")
  wid = sid * NC + cid

  _zero_fill(ones_v, CH, F)
  _zero_acc_slice(ones_v, hist, sid)
  plsc.subcore_barrier()

  one = jnp.ones((16,), jnp.float32)

  def fill_ones(i, _):
    for l in range(F // 16):
      ones_v[i, pl.ds(l * 16, 16)] = one
    return 0

  lax.fori_loop(0, CH, fill_ones, 0)

  def body(j, _):
    pltpu.sync_copy(dst_hbm.at[wid * CPW + j], idxb)
    pltpu.sync_copy(ones_v, hist.at[idxb], add=True)
    return 0

  lax.fori_loop(0, CPW, body, 0)
  plsc.subcore_barrier()
  pltpu.sync_copy(
      hist.at[pl.ds(sid * RPT, RPT)],
      out_hbm.at[cid, pl.ds(sid * RPT, RPT)],
  )


@functools.partial(
    pl.kernel,
    out_type=jax.ShapeDtypeStruct((NC, NPAD, F), jnp.float32),
    mesh=_mesh,
    scratch_types=[
        pltpu.VMEM((CPW // 2, CH), jnp.int32),
        pltpu.VMEM((CPW // 2, CH), jnp.int32),
        pltpu.VMEM((CH, F), jnp.float32),
        pltpu.VMEM((CH, F), jnp.float32),
        pltpu.VMEM_SHARED((NPAD, F), jnp.float32),
        pltpu.SemaphoreType.DMA,
        pltpu.SemaphoreType.DMA,
    ],
)
def _hop_kernel(u_hbm, src_hbm, dst_hbm, out_hbm, srcb, dstb, bufa, bufb,
                acc, sema, semb):
  cid = lax.axis_index("c")
  sid = lax.axis_index("s")
  wid = sid * NC + cid
  cpw_c = jnp.where(cid == 0, CPW0, CPW1)
  wbase = sid * cpw_c + cid * (NS * CPW0)
  hcpw = CPW // 2

  _zero_fill(bufa, CH, F)
  _zero_acc_slice(bufa, acc, sid)
  plsc.subcore_barrier()

  def half(hh, _):
    hbase = wbase + hh * hcpw
    nch = jnp.minimum(cpw_c - hh * hcpw, hcpw)
    pltpu.sync_copy(src_hbm.at[pl.ds(hbase, hcpw)], srcb)
    pltpu.sync_copy(dst_hbm.at[pl.ds(hbase, hcpw)], dstb)
    pltpu.async_copy(u_hbm.at[srcb.at[0]], bufa, sema)

    def body(jj, _):
      j0 = jj * 2
      j1 = j0 + 1
      pltpu.make_async_copy(u_hbm.at[srcb.at[j0]], bufa, sema).wait()
      pltpu.async_copy(u_hbm.at[srcb.at[j1]], bufb, semb)
      pltpu.sync_copy(bufa, acc.at[dstb.at[j0]], add=True)
      pltpu.make_async_copy(u_hbm.at[srcb.at[j1]], bufb, semb).wait()

      @pl.when(j1 + 1 < nch)
      def _():
        pltpu.async_copy(u_hbm.at[srcb.at[j1 + 1]], bufa, sema)

      pltpu.sync_copy(bufb, acc.at[dstb.at[j1]], add=True)
      return 0

    lax.fori_loop(0, nch // 2, body, 0)
    return 0

  nhalves = (cpw_c + hcpw - 1) // hcpw
  lax.fori_loop(0, nhalves, half, 0)
  plsc.subcore_barrier()
  pltpu.sync_copy(
      acc.at[pl.ds(sid * RPT, RPT)],
      out_hbm.at[cid, pl.ds(sid * RPT, RPT)],
  )


_BLK = 512
_NBLK = NPAD // _BLK


def _prep_body(hist_ref, x_ref, u0_ref, dinv_ref, dinv2_ref):
  i = pl.program_id(0)
  deg = hist_ref[0, :, 0:1] + hist_ref[1, :, 0:1] + 1.0
  rows = lax.broadcasted_iota(jnp.int32, (_BLK, 1), 0) + i * _BLK
  dinv = jnp.where(rows < N, lax.rsqrt(deg), 0.0)
  dinv_ref[...] = dinv
  dinv2_ref[...] = dinv * dinv
  u0_ref[...] = x_ref[...] * dinv


def _combine_body(p_ref, u_ref, dinv2_ref, out_ref):
  out_ref[...] = dinv2_ref[...] * (p_ref[0] + p_ref[1] + u_ref[...])


def _pass1_body(p_ref, u_ref, dinv_ref, wt_ref, b_ref, batch_ref, z_ref,
                stats_ref, segz_ref, cnt_ref, ssum, sseg, scnt):
  i = pl.program_id(0)

  @pl.when(i == 0)
  def _():
    ssum[...] = jnp.zeros_like(ssum)
    sseg[...] = jnp.zeros_like(sseg)
    scnt[...] = jnp.zeros_like(scnt)

  hprop = dinv_ref[...] * (p_ref[0] + p_ref[1] + u_ref[...])
  z = jnp.dot(hprop, wt_ref[...], preferred_element_type=jnp.float32)
  z = jnp.maximum(z + b_ref[...], 0.0)
  rows = lax.broadcasted_iota(jnp.int32, (_BLK, 1), 0) + i * _BLK
  zm = jnp.where(rows < N, z, 0.0)
  z_ref[...] = zm
  ssum[0:1, :] += jnp.sum(zm, axis=0, keepdims=True)
  ssum[1:2, :] += jnp.sum(zm * zm, axis=0, keepdims=True)
  gids = lax.broadcasted_iota(jnp.int32, (G, _BLK), 0)
  onehot = (batch_ref[0] == gids).astype(jnp.float32)
  sseg[...] += jnp.dot(onehot, zm, preferred_element_type=jnp.float32)
  scnt[...] += jnp.sum(onehot, axis=1, keepdims=True)

  @pl.when(i == _NBLK - 1)
  def _():
    stats_ref[...] = ssum[...]
    segz_ref[...] = sseg[...]
    cnt_ref[...] = scnt[...]


def _pass2_body(z_ref, stats_ref, segz_ref, cnt_ref, gamma_ref, beta_ref,
                h_ref, grep_ref):
  i = pl.program_id(0)
  mean = stats_ref[0:1, :] * (1.0 / N)
  ex2 = stats_ref[1:2, :] * (1.0 / N)
  var = ex2 - mean * mean
  a = gamma_ref[...] * lax.rsqrt(var + BN_EPS)
  c = beta_ref[...] - mean * a
  h_ref[...] = z_ref[...] * a + c

  @pl.when(i == 0)
  def _():
    grep_ref[...] = segz_ref[...] * a + cnt_ref[...] * c


def _tc_call(body, out_shape, in_specs, out_specs, scratch_shapes=()):
  return pl.pallas_call(
      body,
      grid=(_NBLK,),
      out_shape=out_shape,
      in_specs=in_specs,
      out_specs=out_specs,
      scratch_shapes=list(scratch_shapes),
  )


def kernel(x, edge_index, batch, W, b, gamma, beta):
  f32 = jnp.float32
  src = edge_index[0]
  dst = edge_index[1]
  padi = jnp.full((EPAD + (CPW // 2) * CH - E,), N, jnp.int32)
  src2d = jnp.concatenate([src, padi]).reshape(-1, CH)
  dst2d = jnp.concatenate([dst, padi]).reshape(-1, CH)
  xp = jnp.zeros((NPAD, F), f32).at[:N].set(x)
  batch3d = (jnp.full((NPAD,), G, jnp.int32).at[:N].set(batch)
             .reshape(_NBLK, 1, _BLK))
  wt = W.T
  b2d = b.reshape(1, F)
  gamma2d = gamma.reshape(1, F)
  beta2d = beta.reshape(1, F)

  hist = _deg_kernel(dst2d)

  row_im = lambda i: (i, 0)

  u0, dinv, dinv2 = _tc_call(
      _prep_body,
      out_shape=(
          jax.ShapeDtypeStruct((NPAD, F), f32),
          jax.ShapeDtypeStruct((NPAD, 1), f32),
          jax.ShapeDtypeStruct((NPAD, 1), f32),
      ),
      in_specs=[
          pl.BlockSpec((NC, _BLK, F), lambda i: (0, i, 0)),
          pl.BlockSpec((_BLK, F), row_im),
      ],
      out_specs=(
          pl.BlockSpec((_BLK, F), row_im),
          pl.BlockSpec((_BLK, 1), row_im),
          pl.BlockSpec((_BLK, 1), row_im),
      ),
  )(hist, xp)

  u = u0
  for _ in range(K - 1):
    p = _hop_kernel(u, src2d, dst2d)
    u = _tc_call(
        _combine_body,
        out_shape=jax.ShapeDtypeStruct((NPAD, F), f32),
        in_specs=[
            pl.BlockSpec((NC, _BLK, F), lambda i: (0, i, 0)),
            pl.BlockSpec((_BLK, F), row_im),
            pl.BlockSpec((_BLK, 1), row_im),
        ],
        out_specs=pl.BlockSpec((_BLK, F), row_im),
    )(p, u, dinv2)

  p = _hop_kernel(u, src2d, dst2d)

  z, stats, segz, cnt = _tc_call(
      _pass1_body,
      out_shape=(
          jax.ShapeDtypeStruct((NPAD, F), f32),
          jax.ShapeDtypeStruct((2, F), f32),
          jax.ShapeDtypeStruct((G, F), f32),
          jax.ShapeDtypeStruct((G, 1), f32),
      ),
      in_specs=[
          pl.BlockSpec((NC, _BLK, F), lambda i: (0, i, 0)),
          pl.BlockSpec((_BLK, F), row_im),
          pl.BlockSpec((_BLK, 1), row_im),
          pl.BlockSpec((F, F), lambda i: (0, 0)),
          pl.BlockSpec((1, F), lambda i: (0, 0)),
          pl.BlockSpec((1, 1, _BLK), lambda i: (i, 0, 0)),
      ],
      out_specs=(
          pl.BlockSpec((_BLK, F), row_im),
          pl.BlockSpec((2, F), lambda i: (0, 0)),
          pl.BlockSpec((G, F), lambda i: (0, 0)),
          pl.BlockSpec((G, 1), lambda i: (0, 0)),
      ),
      scratch_shapes=[
          pltpu.VMEM((2, F), f32),
          pltpu.VMEM((G, F), f32),
          pltpu.VMEM((G, 1), f32),
      ],
  )(p, u, dinv, wt, b2d, batch3d)

  h_full, grep = _tc_call(
      _pass2_body,
      out_shape=(
          jax.ShapeDtypeStruct((NPAD, F), f32),
          jax.ShapeDtypeStruct((G, F), f32),
      ),
      in_specs=[
          pl.BlockSpec((_BLK, F), row_im),
          pl.BlockSpec((2, F), lambda i: (0, 0)),
          pl.BlockSpec((G, F), lambda i: (0, 0)),
          pl.BlockSpec((G, 1), lambda i: (0, 0)),
          pl.BlockSpec((1, F), lambda i: (0, 0)),
          pl.BlockSpec((1, F), lambda i: (0, 0)),
      ],
      out_specs=(
          pl.BlockSpec((_BLK, F), row_im),
          pl.BlockSpec((G, F), lambda i: (0, 0)),
      ),
  )(z, stats, segz, cnt, gamma2d, beta2d)

  return (grep, h_full[:N])


# speedup vs baseline: 1.3110x; 1.3110x over previous
import functools

import jax
import jax.numpy as jnp
from jax import lax
from jax.experimental import pallas as pl
from jax.experimental.pallas import tpu as pltpu
from jax.experimental.pallas import tpu_sc as plsc

N = 10000
E = 320000
F = 128
G = 32
K = 3
BN_EPS = 1e-5

NC, NS = 2, 16
NW = NC * NS
NPAD = 10240
CH = 128
CPW = 80
CPW0 = 128
CPW1 = 32
EPAD = NW * CPW * CH
RPT = NPAD // NS

_mesh = plsc.VectorSubcoreMesh(core_axis_name="c", subcore_axis_name="s")


def _zero_fill(buf, nrows, ncols):
  zero = jnp.zeros((16,), jnp.float32)

  def row(i, _):
    for l in range(ncols // 16):
      buf[i, pl.ds(l * 16, 16)] = zero
    return 0

  lax.fori_loop(0, nrows, row, 0)


def _zero_acc_slice(zbuf, acc, sid):
  full, rem = RPT // CH, RPT % CH
  for t in range(full):
    pltpu.sync_copy(zbuf, acc.at[pl.ds(sid * RPT + t * CH, CH)])
  if rem:
    pltpu.sync_copy(zbuf.at[pl.ds(0, rem)],
                    acc.at[pl.ds(sid * RPT + full * CH, rem)])


@functools.partial(
    pl.kernel,
    out_type=jax.ShapeDtypeStruct((NC, NPAD, F), jnp.float32),
    mesh=_mesh,
    scratch_types=[
        pltpu.VMEM((CH,), jnp.int32),
        pltpu.VMEM((CH, F), jnp.float32),
        pltpu.VMEM_SHARED((NPAD, F), jnp.float32),
    ],
)
def _deg_kernel(dst_hbm, out_hbm, idxb, ones_v, hist):
  cid = lax.axis_index("c")
  sid = lax.axis_index("s")
  wid = sid * NC + cid

  _zero_fill(ones_v, CH, F)
  _zero_acc_slice(ones_v, hist, sid)
  plsc.subcore_barrier()

  one = jnp.ones((16,), jnp.float32)

  def fill_ones(i, _):
    for l in range(F // 16):
      ones_v[i, pl.ds(l * 16, 16)] = one
    return 0

  lax.fori_loop(0, CH, fill_ones, 0)

  def body(j, _):
    pltpu.sync_copy(dst_hbm.at[wid * CPW + j], idxb)
    pltpu.sync_copy(ones_v, hist.at[idxb], add=True)
    return 0

  lax.fori_loop(0, CPW, body, 0)
  plsc.subcore_barrier()
  pltpu.sync_copy(
      hist.at[pl.ds(sid * RPT, RPT)],
      out_hbm.at[cid, pl.ds(sid * RPT, RPT)],
  )


@functools.partial(
    pl.kernel,
    out_type=jax.ShapeDtypeStruct((NC, NPAD, F), jnp.float32),
    mesh=_mesh,
    scratch_types=[
        pltpu.VMEM((CPW // 2, CH), jnp.int32),
        pltpu.VMEM((CPW // 2, CH), jnp.int32),
        pltpu.VMEM((CH, F), jnp.float32),
        pltpu.VMEM((CH, F), jnp.float32),
        pltpu.VMEM_SHARED((NPAD, F), jnp.float32),
        pltpu.SemaphoreType.DMA,
        pltpu.SemaphoreType.DMA,
    ],
)
def _hop_kernel(u_hbm, src_hbm, dst_hbm, out_hbm, srcb, dstb, bufa, bufb,
                acc, sema, semb):
  cid = lax.axis_index("c")
  sid = lax.axis_index("s")
  wid = sid * NC + cid
  cpw_c = jnp.where(cid == 0, CPW0, CPW1)
  wbase = sid * cpw_c + cid * (NS * CPW0)
  hcpw = CPW // 2

  _zero_fill(bufa, CH, F)
  _zero_acc_slice(bufa, acc, sid)
  plsc.subcore_barrier()

  def half(hh, _):
    hbase = wbase + hh * hcpw
    nch = jnp.minimum(cpw_c - hh * hcpw, hcpw)
    pltpu.sync_copy(src_hbm.at[pl.ds(hbase, hcpw)], srcb)
    pltpu.sync_copy(dst_hbm.at[pl.ds(hbase, hcpw)], dstb)
    pltpu.async_copy(u_hbm.at[srcb.at[0]], bufa, sema)

    def body(jj, _):
      j0 = jj * 2
      j1 = j0 + 1
      pltpu.make_async_copy(u_hbm.at[srcb.at[j0]], bufa, sema).wait()
      pltpu.async_copy(u_hbm.at[srcb.at[j1]], bufb, semb)
      pltpu.sync_copy(bufa, acc.at[dstb.at[j0]], add=True)
      pltpu.make_async_copy(u_hbm.at[srcb.at[j1]], bufb, semb).wait()

      @pl.when(j1 + 1 < nch)
      def _():
        pltpu.async_copy(u_hbm.at[srcb.at[j1 + 1]], bufa, sema)

      pltpu.sync_copy(bufb, acc.at[dstb.at[j1]], add=True)
      return 0

    lax.fori_loop(0, nch // 2, body, 0)
    return 0

  nhalves = (cpw_c + hcpw - 1) // hcpw
  lax.fori_loop(0, nhalves, half, 0)
  plsc.subcore_barrier()
  pltpu.sync_copy(
      acc.at[pl.ds(sid * RPT, RPT)],
      out_hbm.at[cid, pl.ds(sid * RPT, RPT)],
  )


_BLK = 512
_NBLK = NPAD // _BLK


def _prep_body(hist_ref, x_ref, u0_ref, dinv_ref, dinv2_ref):
  i = pl.program_id(0)
  deg = hist_ref[0, :, 0:1] + hist_ref[1, :, 0:1] + 1.0
  rows = lax.broadcasted_iota(jnp.int32, (_BLK, 1), 0) + i * _BLK
  dinv = jnp.where(rows < N, lax.rsqrt(deg), 0.0)
  dinv_ref[...] = dinv
  dinv2_ref[...] = dinv * dinv
  u0_ref[...] = x_ref[...] * dinv


def _combine_body(p_ref, u_ref, dinv2_ref, out_ref):
  out_ref[...] = dinv2_ref[...] * (p_ref[0] + p_ref[1] + u_ref[...])


def _pass1_body(p_ref, u_ref, dinv_ref, wt_ref, b_ref, batch_ref, z_ref,
                stats_ref, segz_ref, cnt_ref, ssum, sseg, scnt):
  i = pl.program_id(0)

  @pl.when(i == 0)
  def _():
    ssum[...] = jnp.zeros_like(ssum)
    sseg[...] = jnp.zeros_like(sseg)
    scnt[...] = jnp.zeros_like(scnt)

  hprop = dinv_ref[...] * (p_ref[0] + p_ref[1] + u_ref[...])
  z = jnp.dot(hprop, wt_ref[...], preferred_element_type=jnp.float32)
  z = jnp.maximum(z + b_ref[...], 0.0)
  rows = lax.broadcasted_iota(jnp.int32, (_BLK, 1), 0) + i * _BLK
  zm = jnp.where(rows < N, z, 0.0)
  z_ref[...] = zm
  ssum[0:1, :] += jnp.sum(zm, axis=0, keepdims=True)
  ssum[1:2, :] += jnp.sum(zm * zm, axis=0, keepdims=True)
  gids = lax.broadcasted_iota(jnp.int32, (G, _BLK), 0)
  onehot = (batch_ref[0] == gids).astype(jnp.float32)
  sseg[...] += jnp.dot(onehot, zm, preferred_element_type=jnp.float32)
  scnt[...] += jnp.sum(onehot, axis=1, keepdims=True)

  @pl.when(i == _NBLK - 1)
  def _():
    stats_ref[...] = ssum[...]
    segz_ref[...] = sseg[...]
    cnt_ref[...] = scnt[...]


def _pass2_body(z_ref, stats_ref, segz_ref, cnt_ref, gamma_ref, beta_ref,
                h_ref, grep_ref):
  i = pl.program_id(0)
  mean = stats_ref[0:1, :] * (1.0 / N)
  ex2 = stats_ref[1:2, :] * (1.0 / N)
  var = ex2 - mean * mean
  a = gamma_ref[...] * lax.rsqrt(var + BN_EPS)
  c = beta_ref[...] - mean * a
  h_ref[...] = z_ref[...] * a + c

  @pl.when(i == 0)
  def _():
    grep_ref[...] = segz_ref[...] * a + cnt_ref[...] * c


def _tc_call(body, out_shape, in_specs, out_specs, scratch_shapes=()):
  return pl.pallas_call(
      body,
      grid=(_NBLK,),
      out_shape=out_shape,
      in_specs=in_specs,
      out_specs=out_specs,
      scratch_shapes=list(scratch_shapes),
  )


def kernel(x, edge_index, batch, W, b, gamma, beta):
  f32 = jnp.float32
  src = edge_index[0]
  dst = edge_index[1]
  padi = jnp.full((EPAD + (CPW // 2) * CH - E,), N, jnp.int32)
  src2d = jnp.concatenate([src, padi]).reshape(-1, CH)
  dst2d = jnp.concatenate([dst, padi]).reshape(-1, CH)
  xp = jnp.zeros((NPAD, F), f32).at[:N].set(x)
  batch3d = (jnp.full((NPAD,), G, jnp.int32).at[:N].set(batch)
             .reshape(_NBLK, 1, _BLK))
  wt = W.T
  b2d = b.reshape(1, F)
  gamma2d = gamma.reshape(1, F)
  beta2d = beta.reshape(1, F)

  hist = _deg_kernel(dst2d)

  row_im = lambda i: (i, 0)

  u0, dinv, dinv2 = _tc_call(
      _prep_body,
      out_shape=(
          jax.ShapeDtypeStruct((NPAD, F), f32),
          jax.ShapeDtypeStruct((NPAD, 1), f32),
          jax.ShapeDtypeStruct((NPAD, 1), f32),
      ),
      in_specs=[
          pl.BlockSpec((NC, _BLK, F), lambda i: (0, i, 0)),
          pl.BlockSpec((_BLK, F), row_im),
      ],
      out_specs=(
          pl.BlockSpec((_BLK, F), row_im),
          pl.BlockSpec((_BLK, 1), row_im),
          pl.BlockSpec((_BLK, 1), row_im),
      ),
  )(hist, xp)

  u = u0
  for _ in range(K - 1):
    p = _hop_kernel(u, src2d, dst2d)
    u = _tc_call(
        _combine_body,
        out_shape=jax.ShapeDtypeStruct((NPAD, F), f32),
        in_specs=[
            pl.BlockSpec((NC, _BLK, F), lambda i: (0, i, 0)),
            pl.BlockSpec((_BLK, F), row_im),
            pl.BlockSpec((_BLK, 1), row_im),
        ],
        out_specs=pl.BlockSpec((_BLK, F), row_im),
    )(p, u, dinv2)

  p = _hop_kernel(u, src2d, dst2d)

  z, stats, segz, cnt = _tc_call(
      _pass1_body,
      out_shape=(
          jax.ShapeDtypeStruct((NPAD, F), f32),
          jax.ShapeDtypeStruct((2, F), f32),
          jax.ShapeDtypeStruct((G, F), f32),
          jax.ShapeDtypeStruct((G, 1), f32),
      ),
      in_specs=[
          pl.BlockSpec((NC, _BLK, F), lambda i: (0, i, 0)),
          pl.BlockSpec((_BLK, F), row_im),
          pl.BlockSpec((_BLK, 1), row_im),
          pl.BlockSpec((F, F), lambda i: (0, 0)),
          pl.BlockSpec((1, F), lambda i: (0, 0)),
          pl.BlockSpec((1, 1, _BLK), lambda i: (i, 0, 0)),
      ],
      out_specs=(
          pl.BlockSpec((_BLK, F), row_im),
          pl.BlockSpec((2, F), lambda i: (0, 0)),
          pl.BlockSpec((G, F), lambda i: (0, 0)),
          pl.BlockSpec((G, 1), lambda i: (0, 0)),
      ),
      scratch_shapes=[
          pltpu.VMEM((2, F), f32),
          pltpu.VMEM((G, F), f32),
          pltpu.VMEM((G, 1), f32),
      ],
  )(p, u, dinv, wt, b2d, batch3d)

  h_full, grep = _tc_call(
      _pass2_body,
      out_shape=(
          jax.ShapeDtypeStruct((NPAD, F), f32),
          jax.ShapeDtypeStruct((G, F), f32),
      ),
      in_specs=[
          pl.BlockSpec((_BLK, F), row_im),
          pl.BlockSpec((2, F), lambda i: (0, 0)),
          pl.BlockSpec((G, F), lambda i: (0, 0)),
          pl.BlockSpec((G, 1), lambda i: (0, 0)),
          pl.BlockSpec((1, F), lambda i: (0, 0)),
          pl.BlockSpec((1, F), lambda i: (0, 0)),
      ],
      out_specs=(
          pl.BlockSpec((_BLK, F), row_im),
          pl.BlockSpec((G, F), lambda i: (0, 0)),
      ),
  )(z, stats, segz, cnt, gamma2d, beta2d)

  return (grep, h_full[:N])
